# baseline (speedup 1.0000x reference)
import functools

import jax
import jax.numpy as jnp
from jax import lax
from jax.experimental import pallas as pl
from jax.experimental.pallas import tpu as pltpu
from jax.experimental.pallas import tpu_sc as plsc

VOCAB = 1000000
DIM = 32
B = 1024
L = 200

NW = 32
ROWS = B * L
W_ROWS = ROWS // NW
CHUNK = 64
N_CHUNKS = W_ROWS // CHUNK
GROUPS = CHUNK // 16
PPAT = L + CHUNK
SLOTS = 4
AHEAD = 3

QROWS = VOCAB // 4
RB = 2000
NBLK = QROWS // RB


def _repack_body(in_ref, out_ref):
    x = in_ref[...]
    out_ref[...] = jnp.concatenate([x[0], x[1], x[2], x[3]], axis=1)


@jax.jit
def _repack(tok4):
    return pl.pallas_call(
        _repack_body,
        grid=(NBLK,),
        in_specs=[pl.BlockSpec((4, RB, DIM), lambda i: (0, i, 0))],
        out_specs=pl.BlockSpec((RB, 128), lambda i: (i, 0)),
        out_shape=jax.ShapeDtypeStruct((QROWS, 128), jnp.float32),
    )(tok4)


def _gather_body(sidx_hbm, lo_hbm, tok_hbm, pos_hbm, out_hbm,
                 sidx_v, lo_v, pp_v, buf_v, outb_v, sem, sem_out):
    wid = lax.axis_index("s") * 2 + lax.axis_index("c")
    base = wid * W_ROWS

    pltpu.sync_copy(sidx_hbm.at[wid], sidx_v)
    pltpu.sync_copy(lo_hbm.at[wid], lo_v)
    pltpu.sync_copy(pos_hbm, pp_v)

    lane = lax.iota(jnp.int32, 16)

    def fire(c):
        pltpu.async_copy(tok_hbm.at[sidx_v.at[c]], buf_v.at[lax.rem(c, SLOTS)], sem)

    for c0 in range(AHEAD):
        fire(c0)

    def chunk_body(c, _):
        slot = lax.rem(c, SLOTS)
        pltpu.make_async_copy(
            tok_hbm.at[sidx_v.at[c]], buf_v.at[slot], sem).wait()

        @pl.when(c + AHEAD < N_CHUNKS)
        def _prefetch():
            fire(c + AHEAD)

        oslot = lax.rem(c, 2)

        def select_group(g, _):
            r16 = lane + g * 16
            lo16 = lo_v[c, pl.ds(g * 16, 16)]
            for col in range(DIM):
                cc = jnp.full((16,), col, jnp.int32)
                v = plsc.load_gather(buf_v.at[slot], [r16, lo16 + col])
                plsc.store_scatter(outb_v.at[oslot], [r16, cc], v)
            return _

        lax.fori_loop(0, GROUPS, select_group, None)

        off32 = lax.rem(base + c * CHUNK, L) * DIM

        def add_pos(g, _):
            pv = pp_v[pl.ds(off32 + g * 16, 16)]
            plsc.addupdate(outb_v.at[oslot, g // 2, pl.ds(0, 16)], pv)
            pv2 = pp_v[pl.ds(off32 + g * 16 + 16, 16)]
            plsc.addupdate(outb_v.at[oslot, g // 2, pl.ds(16, 16)], pv2)
            return _

        lax.fori_loop(0, CHUNK, lambda g, _: add_pos(2 * g, _), None)

        pltpu.async_copy(
            outb_v.at[oslot],
            out_hbm.at[pl.ds(base + c * CHUNK, CHUNK)], sem_out).wait()
        return _

    lax.fori_loop(0, N_CHUNKS, chunk_body, None)


@jax.jit
def _lookup(sidx2, lo3, tok128, pos_pat):
    mesh = plsc.VectorSubcoreMesh(core_axis_name="c", subcore_axis_name="s")
    f = functools.partial(
        pl.kernel,
        mesh=mesh,
        out_type=jax.ShapeDtypeStruct((ROWS, DIM), jnp.float32),
        scratch_types=[
            pltpu.VMEM((N_CHUNKS, CHUNK), jnp.int32),
            pltpu.VMEM((N_CHUNKS, CHUNK), jnp.int32),
            pltpu.VMEM((PPAT * DIM,), jnp.float32),
            pltpu.VMEM((SLOTS, CHUNK, 128), jnp.float32),
            pltpu.VMEM((2, CHUNK, DIM), jnp.float32),
            pltpu.SemaphoreType.DMA,
            pltpu.SemaphoreType.DMA,
        ],
        compiler_params=pltpu.CompilerParams(
            use_tc_tiling_on_sc=True, needs_layout_passes=False),
    )(_gather_body)
    return f(sidx2, lo3, tok128, pos_pat)


def kernel(indices, token_table, pos_table):
    idx = indices.astype(jnp.int32)
    sidx2 = (idx % QROWS).reshape(NW, N_CHUNKS, CHUNK)
    lo3 = ((idx // QROWS) * DIM).reshape(NW, N_CHUNKS, CHUNK)
    tok4 = token_table.reshape(4, QROWS, DIM)
    tok128 = _repack(tok4)
    pos = pos_table[:L]
    pos_pat = jnp.concatenate([pos, pos[:PPAT - L]], axis=0).reshape(-1)
    out = _lookup(sidx2, lo3, tok128, pos_pat)
    return out.reshape(B, L, DIM)


# speedup vs baseline: 1.2082x; 1.2082x over previous
import functools

import jax
import jax.numpy as jnp
from jax import lax
from jax.experimental import pallas as pl
from jax.experimental.pallas import tpu as pltpu
from jax.experimental.pallas import tpu_sc as plsc

VOCAB = 1000000
DIM = 32
B = 1024
L = 200

NW = 32
ROWS = B * L
W_ROWS = ROWS // NW
S_ROWS = 100
N_STREAMS = W_ROWS // S_ROWS
CHUNK = 1600
S_PER_CHUNK = CHUNK // S_ROWS
N_CHUNKS = W_ROWS // CHUNK
REPS = CHUNK // L


def _body(idx_hbm, tok_hbm, pos_hbm, out_hbm, idx_v, pos_v, buf_v, sem):
    wid = lax.axis_index("s") * 2 + lax.axis_index("c")
    base = wid * W_ROWS

    pltpu.sync_copy(idx_hbm.at[wid], idx_v)
    pltpu.sync_copy(pos_hbm, pos_v)

    for c in range(N_CHUNKS):
        copies = [
            pltpu.async_copy(
                tok_hbm.at[idx_v.at[c * S_PER_CHUNK + j]],
                buf_v.at[pl.ds(j * S_ROWS, S_ROWS)],
                sem,
            )
            for j in range(S_PER_CHUNK)
        ]
        for cp in copies:
            cp.wait()

        def add_pos(l, _):
            p0 = pos_v[l, pl.ds(0, 16)]
            p1 = pos_v[l, pl.ds(16, 16)]
            for q in range(REPS):
                plsc.addupdate(buf_v.at[q * L + l, pl.ds(0, 16)], p0)
                plsc.addupdate(buf_v.at[q * L + l, pl.ds(16, 16)], p1)
            return _

        lax.fori_loop(0, L, add_pos, None)

        pltpu.sync_copy(buf_v, out_hbm.at[pl.ds(base + c * CHUNK, CHUNK)])


@jax.jit
def _lookup(idx3, tok, pos):
    mesh = plsc.VectorSubcoreMesh(core_axis_name="c", subcore_axis_name="s")
    f = functools.partial(
        pl.kernel,
        mesh=mesh,
        out_type=jax.ShapeDtypeStruct((ROWS, DIM), jnp.float32),
        scratch_types=[
            pltpu.VMEM((N_STREAMS, S_ROWS), jnp.int32),
            pltpu.VMEM((L, DIM), jnp.float32),
            pltpu.VMEM((CHUNK, DIM), jnp.float32),
            pltpu.SemaphoreType.DMA,
        ],
        compiler_params=pltpu.CompilerParams(use_tc_tiling_on_sc=False),
    )(_body)
    return f(idx3, tok, pos)


def kernel(indices, token_table, pos_table):
    idx3 = indices.astype(jnp.int32).reshape(NW, N_STREAMS, S_ROWS)
    out = _lookup(idx3, token_table, pos_table[:L])
    return out.reshape(B, L, DIM)
